# baseline (speedup 1.0000x reference)
import functools

import jax
import jax.numpy as jnp
from jax import lax
from jax.experimental import pallas as pl
from jax.experimental.pallas import tpu as pltpu
from jax.experimental.pallas import tpu_sc as plsc

D_MODEL = 4096
N_EXP = 64
TOPK = 8
BT = 1024

SC_NC = 2
SC_NS = 16
SC_NW = SC_NC * SC_NS
SC_CH = 512
SC_SG = 64
SC_L = 16
SC_EU = 2


def _mm_body(x_ref, w_ref, logits_ref, psum_ref, pacc):
    i = pl.program_id(0)
    nb = pl.num_programs(0)

    logits = lax.dot_general(
        x_ref[...], w_ref[...], (((1,), (1,)), ((), ())),
        preferred_element_type=jnp.float32)
    logits_ref[...] = logits.T

    rowmax = jnp.max(logits, axis=1, keepdims=True)
    p = jnp.exp(logits - rowmax)
    probs = p * (1.0 / jnp.sum(p, axis=1, keepdims=True))
    ppart = jnp.sum(probs, axis=0, keepdims=True)

    @pl.when(i == 0)
    def _init():
        pacc[...] = ppart

    @pl.when(i > 0)
    def _acc():
        pacc[...] += ppart

    @pl.when(i == nb - 1)
    def _fin():
        psum_ref[...] = pacc[...]


def _trunc(k):
    return lax.bitcast_convert_type(
        lax.bitcast_convert_type(k, jnp.int32) & jnp.int32(~63), jnp.float32)


def _packed_key(col, e):
    return plsc.bitcast(
        (plsc.bitcast(col, jnp.int32) & jnp.int32(~63))
        | (jnp.int32(63) - e), jnp.float32)


def _sc_body(logits_hbm, gates_hbm, idx_hbm, gsum_hbm, l_v, g_v, i_v, acc_v):
    rpw = logits_hbm.shape[1] // SC_NW
    wid = lax.axis_index("s") * SC_NC + lax.axis_index("c")
    row0 = wid * rpw
    iota = lax.broadcasted_iota(jnp.int32, (SC_L,), 0)
    ngrp = SC_SG // SC_L

    def _zero(e, _):
        acc_v[pl.ds(e * SC_L, SC_L)] = jnp.zeros((SC_L,), jnp.float32)
        return 0

    lax.fori_loop(0, N_EXP, _zero, 0)

    def _chunk(ci, _):
        r0 = row0 + ci * SC_CH
        pltpu.sync_copy(logits_hbm.at[:, pl.ds(r0, SC_CH)], l_v)

        zero = jnp.zeros((SC_L,), jnp.float32)

        def _zfill(t, _):
            for u in range(8):
                g_v[pl.ds((t * 8 + u) * SC_L, SC_L)] = zero
            return 0

        lax.fori_loop(0, SC_CH * N_EXP // SC_L // 8, _zfill, 0)

        def _sgroup(si, _):
            lrs = [si * SC_SG + g * SC_L for g in range(ngrp)]
            rows64 = [(lr + iota) * N_EXP for lr in lrs]
            rows8 = [(lr + iota) * TOPK for lr in lrs]

            def _e_scan(eo, carry):
                regs = list(carry)
                for u in range(SC_EU):
                    e = eo * SC_EU + u
                    for g in range(ngrp):
                        col = l_v[e, pl.ds(lrs[g], SC_L)]
                        cur = _packed_key(col, e)
                        for j in range(TOPK):
                            hi = jnp.maximum(regs[g * TOPK + j], cur)
                            cur = jnp.minimum(regs[g * TOPK + j], cur)
                            regs[g * TOPK + j] = hi
                return tuple(regs)

            neg = jnp.full((SC_L,), -jnp.inf, jnp.float32)
            regs = lax.fori_loop(0, N_EXP // SC_EU, _e_scan,
                                 tuple([neg] * (ngrp * TOPK)))

            for g in range(ngrp):
                m = _trunc(regs[g * TOPK])
                ps = []
                d = jnp.zeros((SC_L,), jnp.float32)
                for j in range(TOPK):
                    pj = jnp.exp(_trunc(regs[g * TOPK + j]) - m)
                    d = d + pj
                    ps.append(pj)
                rs = 1.0 / d
                for j in range(TOPK):
                    ij = jnp.int32(63) - (
                        lax.bitcast_convert_type(regs[g * TOPK + j],
                                                 jnp.int32) & jnp.int32(63))
                    plsc.store_scatter(i_v, [rows8[g] + j], ij)
                    gej = ps[j] * rs
                    plsc.store_scatter(g_v, [rows64[g] + ij], gej)
                    plsc.addupdate_scatter(acc_v, [ij * SC_L + iota], gej)
            return 0

        lax.fori_loop(0, SC_CH // SC_SG, _sgroup, 0)
        pltpu.sync_copy(g_v, gates_hbm.at[pl.ds(r0 * N_EXP, SC_CH * N_EXP)])
        pltpu.sync_copy(i_v, idx_hbm.at[pl.ds(r0 * TOPK, SC_CH * TOPK)])
        return 0

    lax.fori_loop(0, rpw // SC_CH, _chunk, 0)
    pltpu.sync_copy(acc_v, gsum_hbm.at[wid])


N_SPLIT = 2


def _mm_call(x_half, w_gate):
    rows = x_half.shape[0]
    nb = rows // BT
    return pl.pallas_call(
        _mm_body,
        grid=(nb,),
        in_specs=[
            pl.BlockSpec((BT, D_MODEL), lambda i: (i, 0)),
            pl.BlockSpec((N_EXP, D_MODEL), lambda i: (0, 0)),
        ],
        out_specs=[
            pl.BlockSpec((N_EXP, BT), lambda i: (0, i)),
            pl.BlockSpec((1, N_EXP), lambda i: (0, 0)),
        ],
        out_shape=[
            jax.ShapeDtypeStruct((N_EXP, rows), jnp.float32),
            jax.ShapeDtypeStruct((1, N_EXP), jnp.float32),
        ],
        scratch_shapes=[
            pltpu.VMEM((1, N_EXP), jnp.float32),
        ],
    )(x_half, w_gate)


def _sc_call(logits_t):
    rows = logits_t.shape[1]
    return functools.partial(
        pl.kernel,
        out_type=[
            jax.ShapeDtypeStruct((rows * N_EXP,), jnp.float32),
            jax.ShapeDtypeStruct((rows * TOPK,), jnp.int32),
            jax.ShapeDtypeStruct((SC_NW, N_EXP * SC_L), jnp.float32),
        ],
        mesh=plsc.VectorSubcoreMesh(core_axis_name="c", subcore_axis_name="s",
                                    num_cores=SC_NC, num_subcores=SC_NS),
        compiler_params=pltpu.CompilerParams(needs_layout_passes=False),
        scratch_types=[
            pltpu.VMEM((N_EXP, SC_CH), jnp.float32),
            pltpu.VMEM((SC_CH * N_EXP,), jnp.float32),
            pltpu.VMEM((SC_CH * TOPK,), jnp.int32),
            pltpu.VMEM((N_EXP * SC_L,), jnp.float32),
        ],
    )(_sc_body)(logits_t)


def kernel(x, w_gate):
    t_rows = x.shape[0]
    half = t_rows // N_SPLIT
    gates_l, idx_l, gsum_tot, psum_tot = [], [], None, None
    for h in range(N_SPLIT):
        logits_t, psum = _mm_call(
            lax.slice_in_dim(x, h * half, (h + 1) * half, axis=0), w_gate)
        gates_h, idx_h, gsum_h = _sc_call(logits_t)
        gates_l.append(gates_h.reshape(half, N_EXP))
        idx_l.append(idx_h.reshape(half, TOPK))
        gsum_tot = gsum_h if gsum_tot is None else gsum_tot + gsum_h
        psum_tot = psum if psum_tot is None else psum_tot + psum
    gates = jnp.concatenate(gates_l, axis=0)
    idx = jnp.concatenate(idx_l, axis=0)

    gcol = jnp.sum(gsum_tot.reshape(SC_NW, N_EXP, SC_L), axis=(0, 2))
    aux = jnp.sum(gcol * psum_tot[0]) * (N_EXP / (float(t_rows) * t_rows))
    return (gates, idx, aux)


# speedup vs baseline: 2.3916x; 2.3916x over previous
import functools

import jax
import jax.numpy as jnp
from jax import lax
from jax.experimental import pallas as pl
from jax.experimental.pallas import tpu as pltpu
from jax.experimental.pallas import tpu_sc as plsc

D_MODEL = 4096
N_EXP = 64
TOPK = 8
BT = 1024

SC_NC = 2
SC_NS = 16
SC_NW = SC_NC * SC_NS
SC_CH = 512
SC_SG = 64
SC_L = 16
SC_EU = 2


def _mm_body(x_ref, w_ref, logits_ref, psum_ref, pacc):
    i = pl.program_id(0)
    nb = pl.num_programs(0)

    logits = lax.dot_general(
        x_ref[...], w_ref[...], (((1,), (1,)), ((), ())),
        preferred_element_type=jnp.float32)
    logits_ref[...] = logits.T

    rowmax = jnp.max(logits, axis=1, keepdims=True)
    p = jnp.exp(logits - rowmax)
    probs = p * (1.0 / jnp.sum(p, axis=1, keepdims=True))
    ppart = jnp.sum(probs, axis=0, keepdims=True)

    @pl.when(i == 0)
    def _init():
        pacc[...] = ppart

    @pl.when(i > 0)
    def _acc():
        pacc[...] += ppart

    @pl.when(i == nb - 1)
    def _fin():
        psum_ref[...] = pacc[...]


def _trunc(k):
    return lax.bitcast_convert_type(
        lax.bitcast_convert_type(k, jnp.int32) & jnp.int32(~63), jnp.float32)


def _packed_key(col, e):
    return plsc.bitcast(
        (plsc.bitcast(col, jnp.int32) & jnp.int32(~63))
        | (jnp.int32(63) - e), jnp.float32)


def _sc_body(logits_hbm, gates_hbm, idx_hbm, gsum_hbm, l_v, g_v, i_v, acc_v):
    rpw = logits_hbm.shape[1] // SC_NW
    wid = lax.axis_index("s") * SC_NC + lax.axis_index("c")
    row0 = wid * rpw
    iota = lax.broadcasted_iota(jnp.int32, (SC_L,), 0)
    ngrp = SC_SG // SC_L

    def _zero(e, _):
        acc_v[pl.ds(e * SC_L, SC_L)] = jnp.zeros((SC_L,), jnp.float32)
        return 0

    lax.fori_loop(0, N_EXP, _zero, 0)

    def _chunk(ci, _):
        r0 = row0 + ci * SC_CH
        pltpu.sync_copy(logits_hbm.at[:, pl.ds(r0, SC_CH)], l_v)

        zero = jnp.zeros((SC_L,), jnp.float32)

        def _zfill(t, _):
            for u in range(8):
                g_v[pl.ds((t * 8 + u) * SC_L, SC_L)] = zero
            return 0

        lax.fori_loop(0, SC_CH * N_EXP // SC_L // 8, _zfill, 0)

        def _sgroup(si, _):
            lrs = [si * SC_SG + g * SC_L for g in range(ngrp)]
            rows64 = [(lr + iota) * N_EXP for lr in lrs]
            rows8 = [(lr + iota) * TOPK for lr in lrs]

            def _e_scan(eo, carry):
                regs = list(carry)
                for u in range(SC_EU):
                    e = eo * SC_EU + u
                    for g in range(ngrp):
                        col = l_v[e, pl.ds(lrs[g], SC_L)]
                        cur = _packed_key(col, e)
                        for j in range(TOPK):
                            hi = jnp.maximum(regs[g * TOPK + j], cur)
                            cur = jnp.minimum(regs[g * TOPK + j], cur)
                            regs[g * TOPK + j] = hi
                return tuple(regs)

            neg = jnp.full((SC_L,), -jnp.inf, jnp.float32)
            regs = lax.fori_loop(0, N_EXP // SC_EU, _e_scan,
                                 tuple([neg] * (ngrp * TOPK)))

            for g in range(ngrp):
                m = _trunc(regs[g * TOPK])
                ps = []
                d = jnp.zeros((SC_L,), jnp.float32)
                for j in range(TOPK):
                    pj = jnp.exp(_trunc(regs[g * TOPK + j]) - m)
                    d = d + pj
                    ps.append(pj)
                rs = 1.0 / d
                for j in range(TOPK):
                    ij = jnp.int32(63) - (
                        lax.bitcast_convert_type(regs[g * TOPK + j],
                                                 jnp.int32) & jnp.int32(63))
                    plsc.store_scatter(i_v, [rows8[g] + j], ij)
                    gej = ps[j] * rs
                    plsc.store_scatter(g_v, [rows64[g] + ij], gej)
                    plsc.addupdate_scatter(acc_v, [ij * SC_L + iota], gej)
            return 0

        lax.fori_loop(0, SC_CH // SC_SG, _sgroup, 0)
        pltpu.sync_copy(g_v, gates_hbm.at[pl.ds(r0 * N_EXP, SC_CH * N_EXP)])
        pltpu.sync_copy(i_v, idx_hbm.at[pl.ds(r0 * TOPK, SC_CH * TOPK)])
        return 0

    lax.fori_loop(0, rpw // SC_CH, _chunk, 0)
    pltpu.sync_copy(acc_v, gsum_hbm.at[wid])


def _mm_call(x_half, w_gate):
    rows = x_half.shape[0]
    nb = rows // BT
    return pl.pallas_call(
        _mm_body,
        grid=(nb,),
        in_specs=[
            pl.BlockSpec((BT, D_MODEL), lambda i: (i, 0)),
            pl.BlockSpec((N_EXP, D_MODEL), lambda i: (0, 0)),
        ],
        out_specs=[
            pl.BlockSpec((N_EXP, BT), lambda i: (0, i)),
            pl.BlockSpec((1, N_EXP), lambda i: (0, 0)),
        ],
        out_shape=[
            jax.ShapeDtypeStruct((N_EXP, rows), jnp.float32),
            jax.ShapeDtypeStruct((1, N_EXP), jnp.float32),
        ],
        scratch_shapes=[
            pltpu.VMEM((1, N_EXP), jnp.float32),
        ],
    )(x_half, w_gate)


def _sc_call(logits_t):
    rows = logits_t.shape[1]
    return functools.partial(
        pl.kernel,
        out_type=[
            jax.ShapeDtypeStruct((rows * N_EXP,), jnp.float32),
            jax.ShapeDtypeStruct((rows * TOPK,), jnp.int32),
            jax.ShapeDtypeStruct((SC_NW, N_EXP * SC_L), jnp.float32),
        ],
        mesh=plsc.VectorSubcoreMesh(core_axis_name="c", subcore_axis_name="s",
                                    num_cores=SC_NC, num_subcores=SC_NS),
        compiler_params=pltpu.CompilerParams(needs_layout_passes=False),
        scratch_types=[
            pltpu.VMEM((N_EXP, SC_CH), jnp.float32),
            pltpu.VMEM((SC_CH * N_EXP,), jnp.float32),
            pltpu.VMEM((SC_CH * TOPK,), jnp.int32),
            pltpu.VMEM((N_EXP * SC_L,), jnp.float32),
        ],
    )(_sc_body)(logits_t)


def kernel(x, w_gate):
    t_rows = x.shape[0]
    logits_t, psum = _mm_call(x, w_gate)
    gates, idx, gsum = _sc_call(logits_t)
    gates = gates.reshape(t_rows, N_EXP)
    idx = idx.reshape(t_rows, TOPK)

    gcol = jnp.sum(gsum.reshape(SC_NW, N_EXP, SC_L), axis=(0, 2))
    aux = jnp.sum(gcol * psum[0]) * (N_EXP / (float(t_rows) * t_rows))
    return (gates, idx, aux)
